# Initial kernel scaffold; baseline (speedup 1.0000x reference)
import jax
import jax.numpy as jnp
from jax.experimental import pallas as pl


def kernel(batch_xs, batch_pos_enc, W0, a_src0, a_dst0, b0, W1, a_src1, a_dst1, b1, W2, a_src2, a_dst2, b2, linW, linb):
    raise NotImplementedError("write your pallas kernel here")


# speedup vs baseline: 741.5677x; 741.5677x over previous
import functools

import jax
import jax.numpy as jnp
import numpy as np
from jax.experimental import pallas as pl
from jax.experimental.pallas import tpu as pltpu

HEADS = 4
HID = 16
OUT = 6
ENC = 16
PROTO = 64
N_COL = 64

G = 8


def _attention_block(h_all, a_s, a_d, mask, eye4):
    asT = jax.lax.dot_general(
        eye4, a_s, (((1,), (1,)), ((), ())), preferred_element_type=jnp.float32)
    outs = []
    for c in range(G):
        r0 = c * 64
        hc = h_all[r0:r0 + 64, :]
        rows = []
        for hd in range(HEADS):
            s = a_d[r0:r0 + 64, hd:hd + 1] + asT[hd:hd + 1, r0:r0 + 64]
            s = jnp.where(s >= 0, s, 0.2 * s)
            m = jnp.max(s, axis=1, keepdims=True)
            p = jnp.exp(s - m)
            den = jnp.sum(p, axis=1, keepdims=True)
            rows.append(p / den)
        attn = jnp.concatenate(rows, axis=0)
        res = jnp.dot(attn, hc, preferred_element_type=jnp.float32)
        oc = (res[0:64] * mask[0:1] + res[64:128] * mask[1:2]
              + res[128:192] * mask[2:3] + res[192:256] * mask[3:4])
        outs.append(oc)
    return jnp.concatenate(outs, axis=0)


def _gnn_kernel(xsT_ref, pos_ref, w0r0_ref, w0rest_ref, w1_ref, w2_ref,
                as0_ref, ad0_ref, as1_ref, ad1_ref, as2_ref, ad2_ref,
                b0_ref, b1_ref, b2_ref, mask16_ref, mask6_ref, eye4_ref,
                linw_ref, linb_ref, mmean_ref, out_ref):
    eye4 = eye4_ref[...]
    hpos = jnp.dot(pos_ref[0], w0rest_ref[...],
                   preferred_element_type=jnp.float32)
    xsT = xsT_ref[0]
    h_parts = []
    for c in range(G):
        xcol = xsT[:, c:c + 1]
        h_parts.append(xcol * w0r0_ref[...] + hpos)
    h0 = jnp.concatenate(h_parts, axis=0)
    a_s = jnp.dot(h0, as0_ref[...], preferred_element_type=jnp.float32)
    a_d = jnp.dot(h0, ad0_ref[...], preferred_element_type=jnp.float32)
    x1 = _attention_block(h0, a_s, a_d, mask16_ref[...], eye4) + b0_ref[...]

    h1 = jnp.dot(x1, w1_ref[...], preferred_element_type=jnp.float32)
    a_s = jnp.dot(h1, as1_ref[...], preferred_element_type=jnp.float32)
    a_d = jnp.dot(h1, ad1_ref[...], preferred_element_type=jnp.float32)
    x2 = _attention_block(h1, a_s, a_d, mask16_ref[...], eye4) + b1_ref[...]

    h2 = jnp.dot(x2, w2_ref[...], preferred_element_type=jnp.float32)
    a_s = jnp.dot(h2, as2_ref[...], preferred_element_type=jnp.float32)
    a_d = jnp.dot(h2, ad2_ref[...], preferred_element_type=jnp.float32)
    x3 = _attention_block(h2, a_s, a_d, mask6_ref[...], eye4) + b2_ref[...]

    y = jnp.dot(x3, linw_ref[...], preferred_element_type=jnp.float32) \
        + linb_ref[...]
    out_ref[...] = jnp.dot(mmean_ref[...], y,
                           preferred_element_type=jnp.float32)


@jax.jit
def kernel(batch_xs, batch_pos_enc, W0, a_src0, a_dst0, b0,
           W1, a_src1, a_dst1, b1, W2, a_src2, a_dst2, b2, linW, linb):
    bs, n_row, n_col = batch_xs.shape
    ncliq = bs * n_row

    xsT = batch_xs.reshape(ncliq // G, G, n_col).transpose(0, 2, 1)

    def amat(a, d):
        return jnp.kron(jnp.eye(HEADS, dtype=jnp.float32),
                        jnp.ones((d, 1), jnp.float32)) * a.reshape(-1, 1)

    as0, ad0 = amat(a_src0, HID), amat(a_dst0, HID)
    as1, ad1 = amat(a_src1, HID), amat(a_dst1, HID)
    as2, ad2 = amat(a_src2, OUT), amat(a_dst2, OUT)
    mask16 = jnp.kron(jnp.eye(HEADS, dtype=jnp.float32),
                      jnp.ones((1, HID), jnp.float32))
    mask6 = jnp.kron(jnp.eye(HEADS, dtype=jnp.float32),
                     jnp.ones((1, OUT), jnp.float32))
    eye4 = jnp.eye(HEADS, dtype=jnp.float32)
    mmean = jnp.kron(jnp.eye(G, dtype=jnp.float32),
                     jnp.full((1, n_col), 1.0 / n_col, jnp.float32))

    grid = (ncliq // G,)
    rep = lambda *shape: pl.BlockSpec(shape, lambda i: (0,) * len(shape))
    out = pl.pallas_call(
        _gnn_kernel,
        grid=grid,
        in_specs=[
            pl.BlockSpec((1, n_col, G), lambda i: (i, 0, 0)),
            pl.BlockSpec((1, n_col, ENC), lambda i: (i // (n_row // G), 0, 0)),
            rep(1, HEADS * HID),
            rep(ENC, HEADS * HID),
            rep(HEADS * HID, HEADS * HID),
            rep(HEADS * HID, HEADS * OUT),
            rep(HEADS * HID, HEADS), rep(HEADS * HID, HEADS),
            rep(HEADS * HID, HEADS), rep(HEADS * HID, HEADS),
            rep(HEADS * OUT, HEADS), rep(HEADS * OUT, HEADS),
            rep(1, HEADS * HID), rep(1, HEADS * HID),
            rep(1, HEADS * OUT),
            rep(HEADS, HEADS * HID), rep(HEADS, HEADS * OUT),
            rep(HEADS, HEADS),
            rep(HEADS * OUT, PROTO), rep(1, PROTO),
            rep(G, G * n_col),
        ],
        out_specs=pl.BlockSpec((G, PROTO), lambda i: (i, 0)),
        out_shape=jax.ShapeDtypeStruct((ncliq, PROTO), jnp.float32),
    )(xsT, batch_pos_enc, W0[0:1, :], W0[1:, :], W1, W2,
      as0, ad0, as1, ad1, as2, ad2,
      b0.reshape(1, -1), b1.reshape(1, -1), b2.reshape(1, -1),
      mask16, mask6, eye4, linW, linb.reshape(1, -1), mmean)

    return out.reshape(bs, n_row, PROTO)


# speedup vs baseline: 2131.4458x; 2.8742x over previous
import jax
import jax.numpy as jnp
from jax.experimental import pallas as pl

HEADS = 4
HID = 16
OUT = 6
ENC = 16
PROTO = 64

G = 8


def _layer(h_ext, f_dim, l1_mask, ones_top, mask, eye4):
    h = h_ext[:, :f_dim]
    a_s = h_ext[:, f_dim:f_dim + HEADS]
    a_d = h_ext[:, f_dim + HEADS:f_dim + 2 * HEADS]
    asT = jax.lax.dot_general(
        eye4, a_s, (((1,), (1,)), ((), ())),
        preferred_element_type=jnp.float32)
    outs = []
    for c in range(G):
        r0 = c * 64
        hc = h[r0:r0 + 64, :]
        adc = a_d[r0:r0 + 64, :]
        ad_tiled = jnp.concatenate([adc, adc, adc, adc], axis=0)
        lhs = jnp.concatenate([ad_tiled * l1_mask, l1_mask], axis=1)
        rhs = jnp.concatenate([ones_top, asT[:, r0:r0 + 64]], axis=0)
        s = jnp.dot(lhs, rhs, preferred_element_type=jnp.float32)
        s = jnp.where(s >= 0, s, 0.2 * s)
        m = jnp.max(s, axis=1, keepdims=True)
        p = jnp.exp(s - m)
        den = jnp.sum(p, axis=1, keepdims=True)
        attn = p / den
        res = jnp.dot(attn, hc, preferred_element_type=jnp.float32)
        oc = (res[0:64] * mask[0:1] + res[64:128] * mask[1:2]
              + res[128:192] * mask[2:3] + res[192:256] * mask[3:4])
        outs.append(oc)
    return jnp.concatenate(outs, axis=0)


def _gnn_kernel(xsT_ref, pos_ref, w0r0_ref, w0rest_ref, w1_ref, w2_ref,
                b0_ref, b1_ref, b2_ref, l1_ref, ones_ref,
                mask16_ref, mask6_ref, eye4_ref,
                linw_ref, linb_ref, mmean_ref, out_ref):
    eye4 = eye4_ref[...]
    l1_mask = l1_ref[...]
    ones_top = ones_ref[...]
    hpos = jnp.dot(pos_ref[0], w0rest_ref[...],
                   preferred_element_type=jnp.float32)
    xsT = xsT_ref[0]
    h_parts = []
    for c in range(G):
        h_parts.append(xsT[:, c:c + 1] * w0r0_ref[...] + hpos)
    h0 = jnp.concatenate(h_parts, axis=0)
    x1 = _layer(h0, HEADS * HID, l1_mask, ones_top, mask16_ref[...], eye4) \
        + b0_ref[...]

    h1 = jnp.dot(x1, w1_ref[...], preferred_element_type=jnp.float32)
    x2 = _layer(h1, HEADS * HID, l1_mask, ones_top, mask16_ref[...], eye4) \
        + b1_ref[...]

    h2 = jnp.dot(x2, w2_ref[...], preferred_element_type=jnp.float32)
    x3 = _layer(h2, HEADS * OUT, l1_mask, ones_top, mask6_ref[...], eye4) \
        + b2_ref[...]

    y = jnp.dot(x3, linw_ref[...], preferred_element_type=jnp.float32) \
        + linb_ref[...]
    out_ref[...] = jnp.dot(mmean_ref[...], y,
                           preferred_element_type=jnp.float32)


@jax.jit
def kernel(batch_xs, batch_pos_enc, W0, a_src0, a_dst0, b0,
           W1, a_src1, a_dst1, b1, W2, a_src2, a_dst2, b2, linW, linb):
    bs, n_row, n_col = batch_xs.shape
    ncliq = bs * n_row

    xsT = batch_xs.reshape(ncliq // G, G, n_col).transpose(0, 2, 1)

    def amat(a, d):
        return jnp.kron(jnp.eye(HEADS, dtype=jnp.float32),
                        jnp.ones((d, 1), jnp.float32)) * a.reshape(-1, 1)

    def wext(w, a_src, a_dst, d):
        return jnp.concatenate(
            [w, w @ amat(a_src, d), w @ amat(a_dst, d)], axis=1)

    W0e = wext(W0, a_src0, a_dst0, HID)
    W1e = wext(W1, a_src1, a_dst1, HID)
    W2e = wext(W2, a_src2, a_dst2, OUT)

    l1_mask = jnp.kron(jnp.eye(HEADS, dtype=jnp.float32),
                       jnp.ones((n_col, 1), jnp.float32))
    ones_top = jnp.ones((HEADS, n_col), jnp.float32)
    mask16 = jnp.kron(jnp.eye(HEADS, dtype=jnp.float32),
                      jnp.ones((1, HID), jnp.float32))
    mask6 = jnp.kron(jnp.eye(HEADS, dtype=jnp.float32),
                     jnp.ones((1, OUT), jnp.float32))
    eye4 = jnp.eye(HEADS, dtype=jnp.float32)
    mmean = jnp.kron(jnp.eye(G, dtype=jnp.float32),
                     jnp.full((1, n_col), 1.0 / n_col, jnp.float32))

    grid = (ncliq // G,)
    rep = lambda *shape: pl.BlockSpec(shape, lambda i: (0,) * len(shape))
    out = pl.pallas_call(
        _gnn_kernel,
        grid=grid,
        in_specs=[
            pl.BlockSpec((1, n_col, G), lambda i: (i, 0, 0)),
            pl.BlockSpec((1, n_col, ENC), lambda i: (i // (n_row // G), 0, 0)),
            rep(1, 2 * HEADS + HEADS * HID),
            rep(ENC, 2 * HEADS + HEADS * HID),
            rep(HEADS * HID, 2 * HEADS + HEADS * HID),
            rep(HEADS * HID, 2 * HEADS + HEADS * OUT),
            rep(1, HEADS * HID), rep(1, HEADS * HID),
            rep(1, HEADS * OUT),
            rep(HEADS * n_col, HEADS),
            rep(HEADS, n_col),
            rep(HEADS, HEADS * HID), rep(HEADS, HEADS * OUT),
            rep(HEADS, HEADS),
            rep(HEADS * OUT, PROTO), rep(1, PROTO),
            rep(G, G * n_col),
        ],
        out_specs=pl.BlockSpec((G, PROTO), lambda i: (i, 0)),
        out_shape=jax.ShapeDtypeStruct((ncliq, PROTO), jnp.float32),
    )(xsT, batch_pos_enc, W0e[0:1, :], W0e[1:, :], W1e, W2e,
      b0.reshape(1, -1), b1.reshape(1, -1), b2.reshape(1, -1),
      l1_mask, ones_top, mask16, mask6, eye4, linW, linb.reshape(1, -1),
      mmean)

    return out.reshape(bs, n_row, PROTO)


# speedup vs baseline: 3052.3841x; 1.4321x over previous
import jax
import jax.numpy as jnp
from jax.experimental import pallas as pl

HEADS = 4
HID = 16
OUT = 6
ENC = 16
PROTO = 64

G = 8


def _layer(h_ext, f_dim, mask, eye4):
    h = h_ext[:, :f_dim]
    a_s = h_ext[:, f_dim:f_dim + HEADS]
    a_d = h_ext[:, f_dim + HEADS:f_dim + 2 * HEADS]
    asT = jax.lax.dot_general(
        eye4, a_s, (((1,), (1,)), ((), ())),
        preferred_element_type=jnp.float32)
    pieces = []
    for c in range(G):
        for hd in range(HEADS):
            dst_bc = jnp.broadcast_to(
                a_d[c * 64:(c + 1) * 64, hd:hd + 1], (64, 64))
            src_bc = jnp.broadcast_to(
                asT[hd:hd + 1, c * 64:(c + 1) * 64], (64, 64))
            pieces.append(dst_bc + src_bc)
    s = jnp.concatenate(pieces, axis=0)
    s = jnp.where(s >= 0, s, 0.2 * s)
    m = jnp.max(s, axis=1, keepdims=True)
    p = jnp.exp(s - m)
    den = jnp.sum(p, axis=1, keepdims=True)
    attn = p / den
    res = jnp.concatenate(
        [jnp.dot(attn[c * 256:(c + 1) * 256], h[c * 64:(c + 1) * 64, :],
                 preferred_element_type=jnp.float32) for c in range(G)],
        axis=0)
    res4 = res.reshape(G, HEADS, 64, f_dim) * mask[None, :, None, :]
    return res4.sum(axis=1).reshape(G * 64, f_dim)


def _gnn_kernel(xsT_ref, pos_ref, w0r0_ref, w0rest_ref, w1_ref, w2_ref,
                b0_ref, b1_ref, b2_ref,
                mask16_ref, mask6_ref, eye4_ref,
                linw_ref, linb_ref, mmean_ref, out_ref):
    eye4 = eye4_ref[...]
    hpos = jnp.dot(pos_ref[0], w0rest_ref[...],
                   preferred_element_type=jnp.float32)
    xsT = xsT_ref[0]
    h_parts = []
    for c in range(G):
        h_parts.append(xsT[:, c:c + 1] * w0r0_ref[...] + hpos)
    h0 = jnp.concatenate(h_parts, axis=0)
    x1 = _layer(h0, HEADS * HID, mask16_ref[...], eye4) + b0_ref[...]

    h1 = jnp.dot(x1, w1_ref[...], preferred_element_type=jnp.float32)
    x2 = _layer(h1, HEADS * HID, mask16_ref[...], eye4) + b1_ref[...]

    h2 = jnp.dot(x2, w2_ref[...], preferred_element_type=jnp.float32)
    x3 = _layer(h2, HEADS * OUT, mask6_ref[...], eye4) + b2_ref[...]

    y = jnp.dot(x3, linw_ref[...], preferred_element_type=jnp.float32) \
        + linb_ref[...]
    out_ref[...] = jnp.dot(mmean_ref[...], y,
                           preferred_element_type=jnp.float32)


@jax.jit
def kernel(batch_xs, batch_pos_enc, W0, a_src0, a_dst0, b0,
           W1, a_src1, a_dst1, b1, W2, a_src2, a_dst2, b2, linW, linb):
    bs, n_row, n_col = batch_xs.shape
    ncliq = bs * n_row

    xsT = batch_xs.reshape(ncliq // G, G, n_col).transpose(0, 2, 1)

    def amat(a, d):
        return jnp.kron(jnp.eye(HEADS, dtype=jnp.float32),
                        jnp.ones((d, 1), jnp.float32)) * a.reshape(-1, 1)

    def wext(w, a_src, a_dst, d):
        return jnp.concatenate(
            [w, w @ amat(a_src, d), w @ amat(a_dst, d)], axis=1)

    W0e = wext(W0, a_src0, a_dst0, HID)
    W1e = wext(W1, a_src1, a_dst1, HID)
    W2e = wext(W2, a_src2, a_dst2, OUT)

    mask16 = jnp.kron(jnp.eye(HEADS, dtype=jnp.float32),
                      jnp.ones((1, HID), jnp.float32))
    mask6 = jnp.kron(jnp.eye(HEADS, dtype=jnp.float32),
                     jnp.ones((1, OUT), jnp.float32))
    eye4 = jnp.eye(HEADS, dtype=jnp.float32)
    mmean = jnp.kron(jnp.eye(G, dtype=jnp.float32),
                     jnp.full((1, n_col), 1.0 / n_col, jnp.float32))

    grid = (ncliq // G,)
    rep = lambda *shape: pl.BlockSpec(shape, lambda i: (0,) * len(shape))
    out = pl.pallas_call(
        _gnn_kernel,
        grid=grid,
        in_specs=[
            pl.BlockSpec((1, n_col, G), lambda i: (i, 0, 0)),
            pl.BlockSpec((1, n_col, ENC), lambda i: (i // (n_row // G), 0, 0)),
            rep(1, 2 * HEADS + HEADS * HID),
            rep(ENC, 2 * HEADS + HEADS * HID),
            rep(HEADS * HID, 2 * HEADS + HEADS * HID),
            rep(HEADS * HID, 2 * HEADS + HEADS * OUT),
            rep(1, HEADS * HID), rep(1, HEADS * HID),
            rep(1, HEADS * OUT),
            rep(HEADS, HEADS * HID), rep(HEADS, HEADS * OUT),
            rep(HEADS, HEADS),
            rep(HEADS * OUT, PROTO), rep(1, PROTO),
            rep(G, G * n_col),
        ],
        out_specs=pl.BlockSpec((G, PROTO), lambda i: (i, 0)),
        out_shape=jax.ShapeDtypeStruct((ncliq, PROTO), jnp.float32),
    )(xsT, batch_pos_enc, W0e[0:1, :], W0e[1:, :], W1e, W2e,
      b0.reshape(1, -1), b1.reshape(1, -1), b2.reshape(1, -1),
      mask16, mask6, eye4, linW, linb.reshape(1, -1), mmean)

    return out.reshape(bs, n_row, PROTO)


# speedup vs baseline: 3752.9109x; 1.2295x over previous
import jax
import jax.numpy as jnp
from jax.experimental import pallas as pl

HEADS = 4
HID = 16
OUT = 6
ENC = 16
PROTO = 64

G = 64


def _layer(h_ext, f_dim, mask, eye4):
    h = h_ext[:, :f_dim]
    a_s = h_ext[:, f_dim:f_dim + HEADS]
    a_d = h_ext[:, f_dim + HEADS:f_dim + 2 * HEADS]
    asT = jax.lax.dot_general(
        eye4, a_s, (((1,), (1,)), ((), ())),
        preferred_element_type=jnp.float32)
    pieces = []
    for c in range(G):
        for hd in range(HEADS):
            dst_bc = jnp.broadcast_to(
                a_d[c * 64:(c + 1) * 64, hd:hd + 1], (64, 64))
            src_bc = jnp.broadcast_to(
                asT[hd:hd + 1, c * 64:(c + 1) * 64], (64, 64))
            pieces.append(dst_bc + src_bc)
    s = jnp.concatenate(pieces, axis=0)
    s = jnp.where(s >= 0, s, 0.2 * s)
    m = jnp.max(s, axis=1, keepdims=True)
    p = jnp.exp(s - m)
    den = jnp.sum(p, axis=1, keepdims=True)
    attn = p / den
    res = jnp.concatenate(
        [jnp.dot(attn[c * 256:(c + 1) * 256], h[c * 64:(c + 1) * 64, :],
                 preferred_element_type=jnp.float32) for c in range(G)],
        axis=0)
    res4 = res.reshape(G, HEADS, 64, f_dim) * mask[None, :, None, :]
    return res4.sum(axis=1).reshape(G * 64, f_dim)


def _gnn_kernel(xsT_ref, pos_ref, w0r0_ref, w0rest_ref, w1_ref, w2_ref,
                b0_ref, b1_ref, b2_ref,
                mask16_ref, mask6_ref, eye4_ref,
                linw_ref, linb_ref, mmean_ref, out_ref):
    eye4 = eye4_ref[...]
    hpos = jnp.dot(pos_ref[0], w0rest_ref[...],
                   preferred_element_type=jnp.float32)
    xsT = xsT_ref[0]
    h_parts = []
    for c in range(G):
        h_parts.append(xsT[:, c:c + 1] * w0r0_ref[...] + hpos)
    h0 = jnp.concatenate(h_parts, axis=0)
    x1 = _layer(h0, HEADS * HID, mask16_ref[...], eye4) + b0_ref[...]

    h1 = jnp.dot(x1, w1_ref[...], preferred_element_type=jnp.float32)
    x2 = _layer(h1, HEADS * HID, mask16_ref[...], eye4) + b1_ref[...]

    h2 = jnp.dot(x2, w2_ref[...], preferred_element_type=jnp.float32)
    x3 = _layer(h2, HEADS * OUT, mask6_ref[...], eye4) + b2_ref[...]

    y = jnp.dot(x3, linw_ref[...], preferred_element_type=jnp.float32) \
        + linb_ref[...]
    out_ref[...] = jnp.dot(mmean_ref[...], y,
                           preferred_element_type=jnp.float32)


@jax.jit
def kernel(batch_xs, batch_pos_enc, W0, a_src0, a_dst0, b0,
           W1, a_src1, a_dst1, b1, W2, a_src2, a_dst2, b2, linW, linb):
    bs, n_row, n_col = batch_xs.shape
    ncliq = bs * n_row

    xsT = batch_xs.reshape(ncliq // G, G, n_col).transpose(0, 2, 1)

    def amat(a, d):
        return jnp.kron(jnp.eye(HEADS, dtype=jnp.float32),
                        jnp.ones((d, 1), jnp.float32)) * a.reshape(-1, 1)

    def wext(w, a_src, a_dst, d):
        return jnp.concatenate(
            [w, w @ amat(a_src, d), w @ amat(a_dst, d)], axis=1)

    W0e = wext(W0, a_src0, a_dst0, HID)
    W1e = wext(W1, a_src1, a_dst1, HID)
    W2e = wext(W2, a_src2, a_dst2, OUT)

    mask16 = jnp.kron(jnp.eye(HEADS, dtype=jnp.float32),
                      jnp.ones((1, HID), jnp.float32))
    mask6 = jnp.kron(jnp.eye(HEADS, dtype=jnp.float32),
                     jnp.ones((1, OUT), jnp.float32))
    eye4 = jnp.eye(HEADS, dtype=jnp.float32)
    mmean = jnp.kron(jnp.eye(G, dtype=jnp.float32),
                     jnp.full((1, n_col), 1.0 / n_col, jnp.float32))

    grid = (ncliq // G,)
    rep = lambda *shape: pl.BlockSpec(shape, lambda i: (0,) * len(shape))
    out = pl.pallas_call(
        _gnn_kernel,
        grid=grid,
        in_specs=[
            pl.BlockSpec((1, n_col, G), lambda i: (i, 0, 0)),
            pl.BlockSpec((1, n_col, ENC), lambda i: (i // (n_row // G), 0, 0)),
            rep(1, 2 * HEADS + HEADS * HID),
            rep(ENC, 2 * HEADS + HEADS * HID),
            rep(HEADS * HID, 2 * HEADS + HEADS * HID),
            rep(HEADS * HID, 2 * HEADS + HEADS * OUT),
            rep(1, HEADS * HID), rep(1, HEADS * HID),
            rep(1, HEADS * OUT),
            rep(HEADS, HEADS * HID), rep(HEADS, HEADS * OUT),
            rep(HEADS, HEADS),
            rep(HEADS * OUT, PROTO), rep(1, PROTO),
            rep(G, G * n_col),
        ],
        out_specs=pl.BlockSpec((G, PROTO), lambda i: (i, 0)),
        out_shape=jax.ShapeDtypeStruct((ncliq, PROTO), jnp.float32),
    )(xsT, batch_pos_enc, W0e[0:1, :], W0e[1:, :], W1e, W2e,
      b0.reshape(1, -1), b1.reshape(1, -1), b2.reshape(1, -1),
      mask16, mask6, eye4, linW, linb.reshape(1, -1), mmean)

    return out.reshape(bs, n_row, PROTO)


# speedup vs baseline: 3996.1168x; 1.0648x over previous
import jax
import jax.numpy as jnp
from jax.experimental import pallas as pl

HEADS = 4
HID = 16
OUT = 6
ENC = 16
PROTO = 64

G = 32


def _layer(h_ext, f_dim, mask, eye4):
    h = h_ext[:, :f_dim]
    a_s = h_ext[:, f_dim:f_dim + HEADS]
    a_d = h_ext[:, f_dim + HEADS:f_dim + 2 * HEADS]
    asT = jax.lax.dot_general(
        eye4, a_s, (((1,), (1,)), ((), ())),
        preferred_element_type=jnp.float32)
    pieces = []
    for c in range(G):
        for hd in range(HEADS):
            dst_bc = jnp.broadcast_to(
                a_d[c * 64:(c + 1) * 64, hd:hd + 1], (64, 64))
            src_bc = jnp.broadcast_to(
                asT[hd:hd + 1, c * 64:(c + 1) * 64], (64, 64))
            pieces.append(dst_bc + src_bc)
    s = jnp.concatenate(pieces, axis=0)
    s = jnp.where(s >= 0, s, 0.2 * s)
    m = jnp.max(s, axis=1, keepdims=True)
    p = jnp.exp(s - m)
    den = jnp.sum(p, axis=1, keepdims=True)
    attn = p / den
    res = jnp.concatenate(
        [jnp.dot(attn[c * 256:(c + 1) * 256], h[c * 64:(c + 1) * 64, :],
                 preferred_element_type=jnp.float32) for c in range(G)],
        axis=0)
    res4 = res.reshape(G, HEADS, 64, f_dim) * mask[None, :, None, :]
    return res4.sum(axis=1).reshape(G * 64, f_dim)


def _gnn_kernel(xsT_ref, pos_ref, w0r0_ref, w0rest_ref, w1_ref, w2_ref,
                b0_ref, b1_ref, b2_ref,
                mask16_ref, mask6_ref, eye4_ref,
                linw_ref, linb_ref, mmean_ref, out_ref):
    eye4 = eye4_ref[...]
    hpos = jnp.dot(pos_ref[0], w0rest_ref[...],
                   preferred_element_type=jnp.float32)
    xsT = xsT_ref[0]
    h_parts = []
    for c in range(G):
        h_parts.append(xsT[:, c:c + 1] * w0r0_ref[...] + hpos)
    h0 = jnp.concatenate(h_parts, axis=0)
    x1 = _layer(h0, HEADS * HID, mask16_ref[...], eye4) + b0_ref[...]

    h1 = jnp.dot(x1, w1_ref[...], preferred_element_type=jnp.float32)
    x2 = _layer(h1, HEADS * HID, mask16_ref[...], eye4) + b1_ref[...]

    h2 = jnp.dot(x2, w2_ref[...], preferred_element_type=jnp.float32)
    x3 = _layer(h2, HEADS * OUT, mask6_ref[...], eye4) + b2_ref[...]

    y = jnp.dot(x3, linw_ref[...], preferred_element_type=jnp.float32) \
        + linb_ref[...]
    out_ref[...] = jnp.dot(mmean_ref[...], y,
                           preferred_element_type=jnp.float32)


@jax.jit
def kernel(batch_xs, batch_pos_enc, W0, a_src0, a_dst0, b0,
           W1, a_src1, a_dst1, b1, W2, a_src2, a_dst2, b2, linW, linb):
    bs, n_row, n_col = batch_xs.shape
    ncliq = bs * n_row

    xsT = batch_xs.reshape(ncliq // G, G, n_col).transpose(0, 2, 1)

    def amat(a, d):
        return jnp.kron(jnp.eye(HEADS, dtype=jnp.float32),
                        jnp.ones((d, 1), jnp.float32)) * a.reshape(-1, 1)

    def wext(w, a_src, a_dst, d):
        return jnp.concatenate(
            [w, w @ amat(a_src, d), w @ amat(a_dst, d)], axis=1)

    W0e = wext(W0, a_src0, a_dst0, HID)
    W1e = wext(W1, a_src1, a_dst1, HID)
    W2e = wext(W2, a_src2, a_dst2, OUT)

    mask16 = jnp.kron(jnp.eye(HEADS, dtype=jnp.float32),
                      jnp.ones((1, HID), jnp.float32))
    mask6 = jnp.kron(jnp.eye(HEADS, dtype=jnp.float32),
                     jnp.ones((1, OUT), jnp.float32))
    eye4 = jnp.eye(HEADS, dtype=jnp.float32)
    mmean = jnp.kron(jnp.eye(G, dtype=jnp.float32),
                     jnp.full((1, n_col), 1.0 / n_col, jnp.float32))

    grid = (ncliq // G,)
    rep = lambda *shape: pl.BlockSpec(shape, lambda i: (0,) * len(shape))
    out = pl.pallas_call(
        _gnn_kernel,
        grid=grid,
        in_specs=[
            pl.BlockSpec((1, n_col, G), lambda i: (i, 0, 0)),
            pl.BlockSpec((1, n_col, ENC), lambda i: (i // (n_row // G), 0, 0)),
            rep(1, 2 * HEADS + HEADS * HID),
            rep(ENC, 2 * HEADS + HEADS * HID),
            rep(HEADS * HID, 2 * HEADS + HEADS * HID),
            rep(HEADS * HID, 2 * HEADS + HEADS * OUT),
            rep(1, HEADS * HID), rep(1, HEADS * HID),
            rep(1, HEADS * OUT),
            rep(HEADS, HEADS * HID), rep(HEADS, HEADS * OUT),
            rep(HEADS, HEADS),
            rep(HEADS * OUT, PROTO), rep(1, PROTO),
            rep(G, G * n_col),
        ],
        out_specs=pl.BlockSpec((G, PROTO), lambda i: (i, 0)),
        out_shape=jax.ShapeDtypeStruct((ncliq, PROTO), jnp.float32),
    )(xsT, batch_pos_enc, W0e[0:1, :], W0e[1:, :], W1e, W2e,
      b0.reshape(1, -1), b1.reshape(1, -1), b2.reshape(1, -1),
      mask16, mask6, eye4, linW, linb.reshape(1, -1), mmean)

    return out.reshape(bs, n_row, PROTO)
